# baseline (speedup 1.0000x reference)
import functools

import jax
import jax.numpy as jnp
from jax.experimental import pallas as pl
from jax.experimental.pallas import tpu as pltpu

N = 10000

BM = 200
PAIR = 2 * BM
NP = N // PAIR
SPLIT = 5
BS = BM // SPLIT


def _bf(x):
    return x.astype(jnp.bfloat16)


def _fused_kernel(a_hbm, x_hbm, b1_ref, w1_ref, b2_ref, w2_ref,
                  w3_ref, b3_ref, o_ref,
                  ab0, ab1, ab2, ab3,
                  xf_ref, xbf_ref, x1f_ref, x1bf_ref, sems, xsem):
    g = pl.program_id(0)
    total = 2 * NP
    bufs = ((ab0, ab1), (ab2, ab3))

    def copies_for(q, set_id):
        cs = []
        for k in range(2):
            for h in range(SPLIT):
                cs.append(pltpu.make_async_copy(
                    a_hbm.at[pl.ds((q % NP) * PAIR + k * BM + h * BS, BS), :],
                    bufs[set_id][k].at[pl.ds(h * BS, BS), :],
                    sems.at[set_id, k, h],
                ))
        return cs

    @pl.when(g == 0)
    def _prime():
        xcopy = pltpu.make_async_copy(x_hbm, xf_ref, xsem)
        xcopy.start()
        for c in copies_for(0, 0):
            c.start()
        xcopy.wait()
        xbf_ref[...] = _bf(xf_ref[...])

    def compute(set_id):
        @pl.when(g + 1 < total)
        def _prefetch():
            for c in copies_for(g + 1, (set_id + 1) % 2):
                c.start()

        for c in copies_for(g, set_id):
            c.wait()
        a0, a1 = bufs[set_id]

        @pl.when(g < NP)
        def _stage1():
            parts = []
            for k, ab in enumerate((a0, a1)):
                agg = jnp.dot(_bf(ab[...]), xbf_ref[...],
                              preferred_element_type=jnp.float32)
                pre = agg - 3.0 * xf_ref[pl.ds(g * PAIR + k * BM, BM), :]
                hh = jnp.dot(_bf(pre), _bf(w1_ref[...]),
                             preferred_element_type=jnp.float32) + b1_ref[...]
                parts.append(jnp.maximum(hh, 0.0))
            x1 = jnp.concatenate(parts, axis=0)
            x1f_ref[pl.ds(g * PAIR, PAIR), :] = x1
            x1bf_ref[pl.ds(g * PAIR, PAIR), :] = _bf(x1)

        @pl.when(g >= NP)
        def _stage2():
            i = g - NP
            parts = []
            for k, ab in enumerate((a0, a1)):
                agg = jnp.dot(_bf(ab[...]), x1bf_ref[...],
                              preferred_element_type=jnp.float32)
                pre = agg - 2.0 * x1f_ref[pl.ds(i * PAIR + k * BM, BM), :]
                hh = jnp.dot(_bf(pre), _bf(w2_ref[...]),
                             preferred_element_type=jnp.float32) + b2_ref[...]
                x2 = jnp.maximum(hh, 0.0)
                parts.append(jnp.dot(_bf(x2), _bf(w3_ref[...]),
                                     preferred_element_type=jnp.float32)
                             + b3_ref[...])
            o_ref[pl.ds(i * PAIR, PAIR), :] = jnp.concatenate(parts, axis=0)

    for set_id in range(2):
        @pl.when(g % 2 == set_id)
        def _(set_id=set_id):
            compute(set_id)


def kernel(A, X, W1, b1, W2, b2, W3, b3):
    D = X.shape[1]
    H1 = W1.shape[1]
    H2 = W2.shape[1]

    return pl.pallas_call(
        _fused_kernel,
        grid=(2 * NP,),
        in_specs=[
            pl.BlockSpec(memory_space=pltpu.MemorySpace.HBM),
            pl.BlockSpec(memory_space=pltpu.MemorySpace.HBM),
            pl.BlockSpec((1, H1), lambda s: (0, 0)),
            pl.BlockSpec((D, H1), lambda s: (0, 0)),
            pl.BlockSpec((1, H2), lambda s: (0, 0)),
            pl.BlockSpec((H1, H2), lambda s: (0, 0)),
            pl.BlockSpec((H2, 1), lambda s: (0, 0)),
            pl.BlockSpec((1, 1), lambda s: (0, 0)),
        ],
        out_specs=pl.BlockSpec((N, 1), lambda s: (0, 0)),
        out_shape=jax.ShapeDtypeStruct((N, 1), jnp.float32),
        scratch_shapes=[
            pltpu.VMEM((BM, N), jnp.float32),
            pltpu.VMEM((BM, N), jnp.float32),
            pltpu.VMEM((BM, N), jnp.float32),
            pltpu.VMEM((BM, N), jnp.float32),
            pltpu.VMEM((N, D), jnp.float32),
            pltpu.VMEM((N, D), jnp.bfloat16),
            pltpu.VMEM((N, H1), jnp.float32),
            pltpu.VMEM((N, H1), jnp.bfloat16),
            pltpu.SemaphoreType.DMA((2, 2, SPLIT)),
            pltpu.SemaphoreType.DMA,
        ],
        compiler_params=pltpu.CompilerParams(
            dimension_semantics=("arbitrary",),
            vmem_limit_bytes=66 * 1024 * 1024,
        ),
    )(A, X, b1.reshape(1, -1), W1, b2.reshape(1, -1), W2, W3,
      b3.reshape(1, 1))


# speedup vs baseline: 1.0006x; 1.0006x over previous
import jax
import jax.numpy as jnp
from jax.experimental import pallas as pl
from jax.experimental.pallas import tpu as pltpu

N = 10000

BM = 200
PAIR = 2 * BM
NP = N // PAIR
SPLIT = 5
BS = BM // SPLIT


def _bf(x):
    return x.astype(jnp.bfloat16)


def _fused_kernel(a_hbm, x_hbm, b1_ref, w1_ref, b2_ref, w2_ref,
                  w3_ref, b3_ref, o_ref,
                  ab0, ab1, ab2, ab3,
                  xf_ref, xbf_ref, x1f_ref, x1bf_ref, sems, xsem):
    g = pl.program_id(0)
    total = 2 * NP
    bufs = ((ab0, ab1), (ab2, ab3))

    def copies_for(q, set_id):
        cs = []
        for k in range(2):
            for h in range(SPLIT):
                cs.append(pltpu.make_async_copy(
                    a_hbm.at[pl.ds((q % NP) * PAIR + k * BM + h * BS, BS), :],
                    bufs[set_id][k].at[pl.ds(h * BS, BS), :],
                    sems.at[set_id, k, h],
                ))
        return cs

    @pl.when(g == 0)
    def _prime():
        xcopy = pltpu.make_async_copy(x_hbm, xf_ref, xsem)
        xcopy.start()
        for c in copies_for(0, 0):
            c.start()
        xcopy.wait()
        xbf_ref[...] = _bf(xf_ref[...])

    def compute(set_id):
        @pl.when(g + 1 < total)
        def _prefetch():
            for c in copies_for(g + 1, (set_id + 1) % 2):
                c.start()

        for c in copies_for(g, set_id):
            c.wait()
        a0, a1 = bufs[set_id]

        @pl.when(g < NP)
        def _stage1():
            parts = []
            for k, ab in enumerate((a0, a1)):
                agg = jnp.dot(_bf(ab[...]), xbf_ref[...],
                              preferred_element_type=jnp.float32)
                pre = agg - 3.0 * xf_ref[pl.ds(g * PAIR + k * BM, BM), :]
                hh = jnp.dot(_bf(pre), _bf(w1_ref[...]),
                             preferred_element_type=jnp.float32) + b1_ref[...]
                parts.append(jnp.maximum(hh, 0.0))
            x1 = jnp.concatenate(parts, axis=0)
            x1f_ref[pl.ds(g * PAIR, PAIR), :] = x1
            x1bf_ref[pl.ds(g * PAIR, PAIR), :] = _bf(x1)

        @pl.when(g >= NP)
        def _stage2():
            i = g - NP
            parts = []
            for k, ab in enumerate((a0, a1)):
                agg = jnp.dot(_bf(ab[...]), x1bf_ref[...],
                              preferred_element_type=jnp.float32)
                pre = agg - 2.0 * x1f_ref[pl.ds(i * PAIR + k * BM, BM), :]
                hh = jnp.dot(_bf(pre), _bf(w2_ref[...]),
                             preferred_element_type=jnp.float32) + b2_ref[...]
                x2 = jnp.maximum(hh, 0.0)
                parts.append(jnp.dot(_bf(x2), _bf(w3_ref[...]),
                                     preferred_element_type=jnp.float32)
                             + b3_ref[...])
            o_ref[pl.ds(i * PAIR, PAIR), :] = jnp.concatenate(parts, axis=0)

    for set_id in range(2):
        @pl.when(g % 2 == set_id)
        def _(set_id=set_id):
            compute(set_id)


def kernel(A, X, W1, b1, W2, b2, W3, b3):
    D = X.shape[1]
    H1 = W1.shape[1]
    H2 = W2.shape[1]

    return pl.pallas_call(
        _fused_kernel,
        grid=(2 * NP,),
        in_specs=[
            pl.BlockSpec(memory_space=pltpu.MemorySpace.HBM),
            pl.BlockSpec(memory_space=pltpu.MemorySpace.HBM),
            pl.BlockSpec((1, H1), lambda s: (0, 0)),
            pl.BlockSpec((D, H1), lambda s: (0, 0)),
            pl.BlockSpec((1, H2), lambda s: (0, 0)),
            pl.BlockSpec((H1, H2), lambda s: (0, 0)),
            pl.BlockSpec((H2, 1), lambda s: (0, 0)),
            pl.BlockSpec((1, 1), lambda s: (0, 0)),
        ],
        out_specs=pl.BlockSpec((N, 1), lambda s: (0, 0)),
        out_shape=jax.ShapeDtypeStruct((N, 1), jnp.float32),
        scratch_shapes=[
            pltpu.VMEM((BM, N), jnp.float32),
            pltpu.VMEM((BM, N), jnp.float32),
            pltpu.VMEM((BM, N), jnp.float32),
            pltpu.VMEM((BM, N), jnp.float32),
            pltpu.VMEM((N, D), jnp.float32),
            pltpu.VMEM((N, D), jnp.bfloat16),
            pltpu.VMEM((N, H1), jnp.float32),
            pltpu.VMEM((N, H1), jnp.bfloat16),
            pltpu.SemaphoreType.DMA((2, 2, SPLIT)),
            pltpu.SemaphoreType.DMA,
        ],
        compiler_params=pltpu.CompilerParams(
            dimension_semantics=("arbitrary",),
            vmem_limit_bytes=66 * 1024 * 1024,
        ),
    )(A, X, b1.reshape(1, -1), W1, b2.reshape(1, -1), W2, W3,
      b3.reshape(1, 1))
